# Initial kernel scaffold; baseline (speedup 1.0000x reference)
import jax
import jax.numpy as jnp
from jax.experimental import pallas as pl


def kernel(inputs, codebook):
    raise NotImplementedError("write your pallas kernel here")


# speedup vs baseline: 1.7916x; 1.7916x over previous
import jax
import jax.numpy as jnp
from jax.experimental import pallas as pl

K_TOTAL = 8192
D_DIM = 32
M_BLOCK = 256
_CHUNKS = ((0, 2816), (2816, 5632), (5632, 8192))
_BIG = 2**30


def _bf16_round(x):
    return x.astype(jnp.bfloat16).astype(jnp.float32)


def _vq_kernel(a_ref, zsq_ref, csq_ref, c_ref, out_ref):
    dots2 = jax.lax.dot_general(
        a_ref[...], c_ref[...],
        dimension_numbers=(((1,), (1,)), ((), ())),
        preferred_element_type=jnp.float32,
    )
    u = jnp.maximum((zsq_ref[...] + dots2) + csq_ref[...], 0.0)
    iota = jax.lax.broadcasted_iota(jnp.int32, u.shape, 1)

    acc_v = None
    acc_i = None
    for lo, hi in _CHUNKS:
        uc = u[:, lo:hi]
        g = jnp.min(uc, axis=1, keepdims=True)
        cand = jnp.where(uc == g, iota[:, lo:hi], _BIG)
        idx = jnp.min(cand, axis=1, keepdims=True)
        m = jnp.sqrt(g)
        if acc_v is None:
            acc_v, acc_i = _bf16_round(m), idx
        else:
            take = m < acc_v
            acc_i = jnp.where(take, idx, acc_i)
            acc_v = _bf16_round(jnp.where(take, m, acc_v))
    out_ref[...] = acc_i


def kernel(inputs, codebook):
    b, h, w, d = inputs.shape
    m = b * h * w
    a = inputs.reshape(m, d) * jnp.float32(-2.0)
    z_sq = jnp.sum(inputs * inputs, axis=-1, keepdims=True).reshape(m, 1)
    c_sq = jnp.sum(codebook * codebook, axis=-1).reshape(1, K_TOTAL)

    grid = (m // M_BLOCK,)
    out = pl.pallas_call(
        _vq_kernel,
        grid=grid,
        in_specs=[
            pl.BlockSpec((M_BLOCK, d), lambda i: (i, 0)),
            pl.BlockSpec((M_BLOCK, 1), lambda i: (i, 0)),
            pl.BlockSpec((1, K_TOTAL), lambda i: (0, 0)),
            pl.BlockSpec((K_TOTAL, d), lambda i: (0, 0)),
        ],
        out_specs=pl.BlockSpec((M_BLOCK, 1), lambda i: (i, 0)),
        out_shape=jax.ShapeDtypeStruct((m, 1), jnp.int32),
    )(a, z_sq, c_sq, codebook)
    return out.reshape(b, h, w)


# speedup vs baseline: 2.0676x; 1.1541x over previous
import jax
import jax.numpy as jnp
from jax.experimental import pallas as pl

K_TOTAL = 8192
D_DIM = 32
M_BLOCK = 512
_CHUNKS = ((0, 2816), (2816, 5632), (5632, 8192))
_BIG = 2**30


def _bf16_round(x):
    return x.astype(jnp.bfloat16).astype(jnp.float32)


def _vq_kernel(a_ref, zsq_ref, csq_ref, c_ref, out_ref):
    dots2 = jax.lax.dot_general(
        a_ref[...], c_ref[...],
        dimension_numbers=(((1,), (1,)), ((), ())),
        preferred_element_type=jnp.float32,
    )
    u = (zsq_ref[...] + dots2) + csq_ref[...]
    iota = jax.lax.broadcasted_iota(jnp.int32, u.shape, 1)

    acc_v = None
    acc_i = None
    for lo, hi in _CHUNKS:
        uc = u[:, lo:hi]
        g = jnp.min(uc, axis=1, keepdims=True)
        cand = jnp.where(uc == g, iota[:, lo:hi], _BIG)
        idx = jnp.min(cand, axis=1, keepdims=True)
        m = jnp.sqrt(jnp.maximum(g, 0.0))
        if acc_v is None:
            acc_v, acc_i = _bf16_round(m), idx
        else:
            take = m < acc_v
            acc_i = jnp.where(take, idx, acc_i)
            acc_v = _bf16_round(jnp.where(take, m, acc_v))
    out_ref[...] = acc_i


def kernel(inputs, codebook):
    b, h, w, d = inputs.shape
    m = b * h * w
    a = inputs.reshape(m, d) * jnp.float32(-2.0)
    z_sq = jnp.sum(inputs * inputs, axis=-1, keepdims=True).reshape(m, 1)
    c_sq = jnp.sum(codebook * codebook, axis=-1).reshape(1, K_TOTAL)

    grid = (m // M_BLOCK,)
    out = pl.pallas_call(
        _vq_kernel,
        grid=grid,
        in_specs=[
            pl.BlockSpec((M_BLOCK, d), lambda i: (i, 0)),
            pl.BlockSpec((M_BLOCK, 1), lambda i: (i, 0)),
            pl.BlockSpec((1, K_TOTAL), lambda i: (0, 0)),
            pl.BlockSpec((K_TOTAL, d), lambda i: (0, 0)),
        ],
        out_specs=pl.BlockSpec((M_BLOCK, 1), lambda i: (i, 0)),
        out_shape=jax.ShapeDtypeStruct((m, 1), jnp.int32),
    )(a, z_sq, c_sq, codebook)
    return out.reshape(b, h, w)


# speedup vs baseline: 2.2856x; 1.1054x over previous
import jax
import jax.numpy as jnp
from jax.experimental import pallas as pl

K_TOTAL = 8192
D_DIM = 32
M_BLOCK = 512
_CHUNKS = ((0, 2816), (2816, 5632), (5632, 8192))
_BIG = 3.0e7


def _bf16_round(x):
    return x.astype(jnp.bfloat16).astype(jnp.float32)


def _vq_kernel(a_ref, zsq_ref, csq_ref, iota_ref, c_ref, out_ref):
    dots2 = jax.lax.dot_general(
        a_ref[...], c_ref[...],
        dimension_numbers=(((1,), (1,)), ((), ())),
        preferred_element_type=jnp.float32,
    )
    u = (zsq_ref[...] + dots2) + csq_ref[...]
    iota = iota_ref[...]

    acc_v = None
    acc_i = None
    for lo, hi in _CHUNKS:
        uc = u[:, lo:hi]
        g = jnp.min(uc, axis=1, keepdims=True)
        cand = jnp.where(uc == g, iota[:, lo:hi], _BIG)
        idx = jnp.min(cand, axis=1, keepdims=True)
        m = jnp.sqrt(jnp.maximum(g, 0.0))
        if acc_v is None:
            acc_v, acc_i = _bf16_round(m), idx
        else:
            take = m < acc_v
            acc_i = jnp.where(take, idx, acc_i)
            acc_v = _bf16_round(jnp.where(take, m, acc_v))
    out_ref[...] = acc_i.astype(jnp.int32)


def kernel(inputs, codebook):
    b, h, w, d = inputs.shape
    m = b * h * w
    a = inputs.reshape(m, d) * jnp.float32(-2.0)
    z_sq = jnp.sum(inputs * inputs, axis=-1, keepdims=True).reshape(m, 1)
    c_sq = jnp.sum(codebook * codebook, axis=-1).reshape(1, K_TOTAL)
    iota = jnp.arange(K_TOTAL, dtype=jnp.float32).reshape(1, K_TOTAL)

    grid = (m // M_BLOCK,)
    out = pl.pallas_call(
        _vq_kernel,
        grid=grid,
        in_specs=[
            pl.BlockSpec((M_BLOCK, d), lambda i: (i, 0)),
            pl.BlockSpec((M_BLOCK, 1), lambda i: (i, 0)),
            pl.BlockSpec((1, K_TOTAL), lambda i: (0, 0)),
            pl.BlockSpec((1, K_TOTAL), lambda i: (0, 0)),
            pl.BlockSpec((K_TOTAL, d), lambda i: (0, 0)),
        ],
        out_specs=pl.BlockSpec((M_BLOCK, 1), lambda i: (i, 0)),
        out_shape=jax.ShapeDtypeStruct((m, 1), jnp.int32),
    )(a, z_sq, c_sq, iota, codebook)
    return out.reshape(b, h, w)


# speedup vs baseline: 2.3339x; 1.0211x over previous
import jax
import jax.numpy as jnp
from jax.experimental import pallas as pl

K_TOTAL = 8192
D_DIM = 32
M_BLOCK = 1024
_CHUNKS = ((0, 2816), (2816, 5632), (5632, 8192))
_BIG = 3.0e7


def _bf16_round(x):
    return x.astype(jnp.bfloat16).astype(jnp.float32)


def _vq_kernel(a_ref, zsq_ref, csq_ref, iota_ref, c_ref, out_ref):
    dots2 = jax.lax.dot_general(
        a_ref[...], c_ref[...],
        dimension_numbers=(((1,), (1,)), ((), ())),
        preferred_element_type=jnp.float32,
    )
    u = (zsq_ref[...] + dots2) + csq_ref[...]
    iota = iota_ref[...]

    acc_v = None
    acc_i = None
    for lo, hi in _CHUNKS:
        uc = u[:, lo:hi]
        g = jnp.min(uc, axis=1, keepdims=True)
        cand = jnp.where(uc == g, iota[:, lo:hi], _BIG)
        idx = jnp.min(cand, axis=1, keepdims=True)
        m = jnp.sqrt(jnp.maximum(g, 0.0))
        if acc_v is None:
            acc_v, acc_i = _bf16_round(m), idx
        else:
            take = m < acc_v
            acc_i = jnp.where(take, idx, acc_i)
            acc_v = _bf16_round(jnp.where(take, m, acc_v))
    out_ref[...] = acc_i.astype(jnp.int32)


def kernel(inputs, codebook):
    b, h, w, d = inputs.shape
    m = b * h * w
    a = inputs.reshape(m, d) * jnp.float32(-2.0)
    z_sq = jnp.sum(inputs * inputs, axis=-1, keepdims=True).reshape(m, 1)
    c_sq = jnp.sum(codebook * codebook, axis=-1).reshape(1, K_TOTAL)
    iota = jnp.arange(K_TOTAL, dtype=jnp.float32).reshape(1, K_TOTAL)

    grid = (m // M_BLOCK,)
    out = pl.pallas_call(
        _vq_kernel,
        grid=grid,
        in_specs=[
            pl.BlockSpec((M_BLOCK, d), lambda i: (i, 0)),
            pl.BlockSpec((M_BLOCK, 1), lambda i: (i, 0)),
            pl.BlockSpec((1, K_TOTAL), lambda i: (0, 0)),
            pl.BlockSpec((1, K_TOTAL), lambda i: (0, 0)),
            pl.BlockSpec((K_TOTAL, d), lambda i: (0, 0)),
        ],
        out_specs=pl.BlockSpec((M_BLOCK, 1), lambda i: (i, 0)),
        out_shape=jax.ShapeDtypeStruct((m, 1), jnp.int32),
    )(a, z_sq, c_sq, iota, codebook)
    return out.reshape(b, h, w)


# speedup vs baseline: 2.3484x; 1.0062x over previous
import jax
import jax.numpy as jnp
from jax.experimental import pallas as pl

K_TOTAL = 8192
D_DIM = 32
M_BLOCK = 2048
_CHUNKS = ((0, 2816), (2816, 5632), (5632, 8192))
_BIG = 3.0e7


def _bf16_round(x):
    return x.astype(jnp.bfloat16).astype(jnp.float32)


def _vq_kernel(a_ref, zsq_ref, csq_ref, iota_ref, c_ref, out_ref):
    dots2 = jax.lax.dot_general(
        a_ref[...], c_ref[...],
        dimension_numbers=(((1,), (1,)), ((), ())),
        preferred_element_type=jnp.float32,
    )
    u = (zsq_ref[...] + dots2) + csq_ref[...]
    iota = iota_ref[...]

    acc_v = None
    acc_i = None
    for lo, hi in _CHUNKS:
        uc = u[:, lo:hi]
        g = jnp.min(uc, axis=1, keepdims=True)
        cand = jnp.where(uc == g, iota[:, lo:hi], _BIG)
        idx = jnp.min(cand, axis=1, keepdims=True)
        m = jnp.sqrt(jnp.maximum(g, 0.0))
        if acc_v is None:
            acc_v, acc_i = _bf16_round(m), idx
        else:
            take = m < acc_v
            acc_i = jnp.where(take, idx, acc_i)
            acc_v = _bf16_round(jnp.where(take, m, acc_v))
    out_ref[...] = acc_i.astype(jnp.int32)


def kernel(inputs, codebook):
    b, h, w, d = inputs.shape
    m = b * h * w
    a = inputs.reshape(m, d) * jnp.float32(-2.0)
    z_sq = jnp.sum(inputs * inputs, axis=-1, keepdims=True).reshape(m, 1)
    c_sq = jnp.sum(codebook * codebook, axis=-1).reshape(1, K_TOTAL)
    iota = jnp.arange(K_TOTAL, dtype=jnp.float32).reshape(1, K_TOTAL)

    grid = (m // M_BLOCK,)
    out = pl.pallas_call(
        _vq_kernel,
        grid=grid,
        in_specs=[
            pl.BlockSpec((M_BLOCK, d), lambda i: (i, 0)),
            pl.BlockSpec((M_BLOCK, 1), lambda i: (i, 0)),
            pl.BlockSpec((1, K_TOTAL), lambda i: (0, 0)),
            pl.BlockSpec((1, K_TOTAL), lambda i: (0, 0)),
            pl.BlockSpec((K_TOTAL, d), lambda i: (0, 0)),
        ],
        out_specs=pl.BlockSpec((M_BLOCK, 1), lambda i: (i, 0)),
        out_shape=jax.ShapeDtypeStruct((m, 1), jnp.int32),
    )(a, z_sq, c_sq, iota, codebook)
    return out.reshape(b, h, w)


# speedup vs baseline: 2.7489x; 1.1705x over previous
import jax
import jax.numpy as jnp
from jax.experimental import pallas as pl

K_TOTAL = 8192
D_DIM = 32
M_BLOCK = 2048
_CHUNKS = ((0, 2816), (2816, 5632), (5632, 8192))
_BIG = 3.0e7


def _bf16_round(x):
    return x.astype(jnp.bfloat16).astype(jnp.float32)


def _vq_kernel(a_ref, zsq_ref, csq_ref, iota_ref, c_ref, out_ref):
    dots2 = jax.lax.dot_general(
        a_ref[...], c_ref[...],
        dimension_numbers=(((1,), (1,)), ((), ())),
        preferred_element_type=jnp.float32,
    )
    u = (zsq_ref[...] + dots2) + csq_ref[...]
    iota = iota_ref[...]

    mb = u.shape[0]
    acc_v = None
    acc_i = None
    for lo, hi in _CHUNKS:
        g = jnp.full((mb, 128), jnp.inf, jnp.float32)
        lidx = jnp.zeros((mb, 128), jnp.float32)
        for j in range(lo, hi, 128):
            uj = u[:, j:j + 128]
            take = uj < g
            lidx = jnp.where(take, iota[:, j:j + 128], lidx)
            g = jnp.minimum(g, uj)
        gm = jnp.min(g, axis=1, keepdims=True)
        cand = jnp.where(g == gm, lidx, _BIG)
        idx = jnp.min(cand, axis=1, keepdims=True)
        m = jnp.sqrt(jnp.maximum(gm, 0.0))
        if acc_v is None:
            acc_v, acc_i = _bf16_round(m), idx
        else:
            take = m < acc_v
            acc_i = jnp.where(take, idx, acc_i)
            acc_v = _bf16_round(jnp.where(take, m, acc_v))
    out_ref[...] = acc_i.astype(jnp.int32)


def kernel(inputs, codebook):
    b, h, w, d = inputs.shape
    m = b * h * w
    a = inputs.reshape(m, d) * jnp.float32(-2.0)
    z_sq = jnp.sum(inputs * inputs, axis=-1, keepdims=True).reshape(m, 1)
    c_sq = jnp.sum(codebook * codebook, axis=-1).reshape(1, K_TOTAL)
    iota = jnp.arange(K_TOTAL, dtype=jnp.float32).reshape(1, K_TOTAL)

    grid = (m // M_BLOCK,)
    out = pl.pallas_call(
        _vq_kernel,
        grid=grid,
        in_specs=[
            pl.BlockSpec((M_BLOCK, d), lambda i: (i, 0)),
            pl.BlockSpec((M_BLOCK, 1), lambda i: (i, 0)),
            pl.BlockSpec((1, K_TOTAL), lambda i: (0, 0)),
            pl.BlockSpec((1, K_TOTAL), lambda i: (0, 0)),
            pl.BlockSpec((K_TOTAL, d), lambda i: (0, 0)),
        ],
        out_specs=pl.BlockSpec((M_BLOCK, 1), lambda i: (i, 0)),
        out_shape=jax.ShapeDtypeStruct((m, 1), jnp.int32),
    )(a, z_sq, c_sq, iota, codebook)
    return out.reshape(b, h, w)


# speedup vs baseline: 2.7771x; 1.0103x over previous
import jax
import jax.numpy as jnp
from jax.experimental import pallas as pl

K_TOTAL = 8192
D_DIM = 32
M_BLOCK = 2048
_CHUNKS = ((0, 2816), (2816, 5632), (5632, 8192))
_BIG = 3.0e7


def _bf16_round(x):
    return x.astype(jnp.bfloat16).astype(jnp.float32)


def _vq_kernel(z_ref, zsq_ref, csq_ref, iota_ref, c_ref, out_ref):
    a = z_ref[...] * jnp.float32(-2.0)
    dots2 = jax.lax.dot_general(
        a, c_ref[...],
        dimension_numbers=(((1,), (1,)), ((), ())),
        preferred_element_type=jnp.float32,
    )
    u = (zsq_ref[...] + dots2) + csq_ref[...]
    iota = iota_ref[...]

    mb = u.shape[0]
    acc_v = None
    acc_i = None
    for lo, hi in _CHUNKS:
        g = jnp.full((mb, 128), jnp.inf, jnp.float32)
        lidx = jnp.zeros((mb, 128), jnp.float32)
        for j in range(lo, hi, 128):
            uj = u[:, j:j + 128]
            take = uj < g
            lidx = jnp.where(take, iota[:, j:j + 128], lidx)
            g = jnp.minimum(g, uj)
        gm = jnp.min(g, axis=1, keepdims=True)
        cand = jnp.where(g == gm, lidx, _BIG)
        idx = jnp.min(cand, axis=1, keepdims=True)
        m = jnp.sqrt(jnp.maximum(gm, 0.0))
        if acc_v is None:
            acc_v, acc_i = _bf16_round(m), idx
        else:
            take = m < acc_v
            acc_i = jnp.where(take, idx, acc_i)
            acc_v = _bf16_round(jnp.where(take, m, acc_v))
    out_ref[...] = acc_i.astype(jnp.int32)


def kernel(inputs, codebook):
    b, h, w, d = inputs.shape
    m = b * h * w
    z = inputs.reshape(m, d)
    z_sq = jnp.sum(inputs * inputs, axis=-1, keepdims=True).reshape(m, 1)
    c_sq = jnp.sum(codebook * codebook, axis=-1).reshape(1, K_TOTAL)
    iota = jnp.arange(K_TOTAL, dtype=jnp.float32).reshape(1, K_TOTAL)

    grid = (m // M_BLOCK,)
    out = pl.pallas_call(
        _vq_kernel,
        grid=grid,
        in_specs=[
            pl.BlockSpec((M_BLOCK, d), lambda i: (i, 0)),
            pl.BlockSpec((M_BLOCK, 1), lambda i: (i, 0)),
            pl.BlockSpec((1, K_TOTAL), lambda i: (0, 0)),
            pl.BlockSpec((1, K_TOTAL), lambda i: (0, 0)),
            pl.BlockSpec((K_TOTAL, d), lambda i: (0, 0)),
        ],
        out_specs=pl.BlockSpec((M_BLOCK, 1), lambda i: (i, 0)),
        out_shape=jax.ShapeDtypeStruct((m, 1), jnp.int32),
    )(z, z_sq, c_sq, iota, codebook)
    return out.reshape(b, h, w)
